# Initial kernel scaffold; baseline (speedup 1.0000x reference)
import jax
import jax.numpy as jnp
from jax.experimental import pallas as pl


def kernel(x, edge_index, edge_attr, W_mean, b_mean, W_std, b_std):
    raise NotImplementedError("write your pallas kernel here")


# speedup vs baseline: 6.1245x; 6.1245x over previous
import functools

import jax
import jax.numpy as jnp
from jax import lax
from jax.experimental import pallas as pl
from jax.experimental.pallas import tpu as pltpu
from jax.experimental.pallas import tpu_sc as plsc

N = 10000
E = 320000
D = 128
NP = 10240
NC = 2
NS = 16
L = 16
CHUNK = 128
NCHUNKS = E // CHUNK
RPS = NP // NS
DEGR = NP // L


def _lin_body(x_ref, w_ref, b_ref, o_ref):
    o_ref[0] = (
        lax.dot_general(
            x_ref[...], w_ref[0], (((1,), (1,)), ((), ())),
            preferred_element_type=jnp.float32,
        )
        + b_ref[0]
    )


def _linear(x_pad, W_all, b_all):
    BM = 2048
    return pl.pallas_call(
        _lin_body,
        grid=(2, NP // BM),
        in_specs=[
            pl.BlockSpec((BM, D), lambda l, i: (i, 0)),
            pl.BlockSpec((1, D, D), lambda l, i: (l, 0, 0)),
            pl.BlockSpec((1, 1, D), lambda l, i: (l, 0, 0)),
        ],
        out_specs=pl.BlockSpec((1, BM, D), lambda l, i: (l, i, 0)),
        out_shape=jax.ShapeDtypeStruct((2, NP, D), jnp.float32),
    )(x_pad, W_all, b_all)


_mesh = plsc.VectorSubcoreMesh(
    core_axis_name="c", subcore_axis_name="s", num_cores=NC, num_subcores=NS
)


@functools.partial(
    pl.kernel,
    out_type=jax.ShapeDtypeStruct((2 * NP, D), jnp.float32),
    mesh=_mesh,
    compiler_params=pltpu.CompilerParams(
        needs_layout_passes=False, use_tc_tiling_on_sc=False
    ),
    scratch_types=[
        pltpu.VMEM_SHARED((DEGR, L), jnp.float32),
        pltpu.VMEM_SHARED((NP, D), jnp.float32),
        pltpu.VMEM((DEGR, L), jnp.float32),
        pltpu.VMEM((5, CHUNK), jnp.int32),
        pltpu.VMEM((4, CHUNK), jnp.int32),
        pltpu.VMEM((1, CHUNK), jnp.int32),
        pltpu.VMEM((1, CHUNK), jnp.int32),
        pltpu.VMEM((1, CHUNK), jnp.int32),
        pltpu.VMEM((CHUNK, D), jnp.float32),
        pltpu.VMEM((CHUNK, D), jnp.float32),
        pltpu.SemaphoreType.DMA,
        pltpu.SemaphoreType.DMA,
    ],
)
def _sc_kernel(
    rows2d, cols2d, radj2d, attr, hflat, zrows, zdeg, idrows, out,
    dcomp_sp, s_sp, dis_v, idr_v, colq_v,
    row_v, col_v, radj_v, hrows_v, attr_v, sem_g, sem_a,
):
    cid = lax.axis_index("c")
    sid = lax.axis_index("s")
    one16 = jnp.ones((L,), jnp.float32)

    pltpu.sync_copy(zdeg, dis_v)
    pltpu.sync_copy(idrows, idr_v)
    pltpu.sync_copy(zdeg.at[pl.ds(0, 40)], dcomp_sp.at[pl.ds(sid * 40, 40)])
    for k in range(5):
        pltpu.sync_copy(zrows, s_sp.at[pl.ds(sid * RPS + k * 128, 128)])
    plsc.subcore_barrier()

    def hgroup(g, carry):
        r = sid * 156 + g * 4
        pltpu.sync_copy(cols2d.at[pl.ds(r, 4)], colq_v)
        for a in range(4):
            for bq in range(8):
                c16 = colq_v[a, pl.ds(bq * 16, 16)]
                hi = lax.shift_right_logical(c16, 4)
                lo = lax.bitwise_and(c16, 15)
                plsc.addupdate_scatter(dis_v, [hi, lo], one16)
        return carry

    lax.fori_loop(0, 39, hgroup, 0)

    @pl.when(sid == 15)
    def _():
        pltpu.sync_copy(cols2d.at[pl.ds(2496, 4)], colq_v)
        for a in range(4):
            for bq in range(8):
                c16 = colq_v[a, pl.ds(bq * 16, 16)]
                hi = lax.shift_right_logical(c16, 4)
                lo = lax.bitwise_and(c16, 15)
                plsc.addupdate_scatter(dis_v, [hi, lo], one16)

    for k in range(5):
        pltpu.sync_copy(
            dis_v.at[pl.ds(k * 128, 128)], dcomp_sp.at[idr_v.at[k]], add=True
        )
    plsc.subcore_barrier()

    pltpu.sync_copy(dcomp_sp, dis_v)

    def newton(i, carry):
        d = dis_v[i]
        ib = plsc.bitcast(d, jnp.int32)
        ib = 0x5F3759DF - lax.shift_right_logical(ib, 1)
        y = plsc.bitcast(ib, jnp.float32)
        y = y * (1.5 - 0.5 * d * y * y)
        y = y * (1.5 - 0.5 * d * y * y)
        y = y * (1.5 - 0.5 * d * y * y)
        dis_v[i] = jnp.where(d > 0.5, y, 0.0)
        return carry

    lax.fori_loop(0, DEGR, newton, 0)

    nch = jnp.where(sid < 4, 157, 156)

    def echunk(m, carry):
        c = sid + 16 * m
        base = pl.multiple_of(c * 128, 128)
        pltpu.sync_copy(rows2d.at[pl.ds(c, 1)], row_v)
        pltpu.sync_copy(cols2d.at[pl.ds(c, 1)], col_v)
        pltpu.sync_copy(radj2d.at[pl.ds(cid * NCHUNKS + c, 1)], radj_v)
        cp = pltpu.async_copy(hflat.at[radj_v.at[0]], hrows_v, sem_g)
        pltpu.async_copy(attr.at[pl.ds(base, 128)], attr_v, sem_a).wait()
        cp.wait()

        def rowloop(jv, carry2):
            r16 = row_v[0, pl.ds(jv * 16, 16)]
            hi = lax.shift_right_logical(r16, 4)
            lo = lax.bitwise_and(r16, 15)
            drv = plsc.load_gather(dis_v, [hi, lo])
            for j in range(16):
                e = jv * 16 + j
                b = jnp.broadcast_to(drv[j], (L,))
                for f in range(8):
                    sl = pl.ds(f * 16, 16)
                    attr_v[e, sl] = (attr_v[e, sl] + hrows_v[e, sl]) * b
            return carry2

        lax.fori_loop(0, 8, rowloop, 0)
        pltpu.sync_copy(attr_v, s_sp.at[col_v.at[0]], add=True)
        return carry

    lax.fori_loop(0, nch, echunk, 0)
    plsc.subcore_barrier()

    for k in range(5):
        r0 = sid * RPS + k * 128
        pltpu.sync_copy(s_sp.at[pl.ds(r0, 128)], hrows_v)
        pltpu.sync_copy(hflat.at[pl.ds(cid * NP + r0, 128)], attr_v)

        def wrow(j, carry):
            node = r0 + j
            hi = jnp.broadcast_to(lax.shift_right_logical(node, 4), (L,))
            lo = jnp.broadcast_to(lax.bitwise_and(node, 15), (L,))
            dn = plsc.load_gather(dis_v, [hi, lo])
            for f in range(8):
                sl = pl.ds(f * 16, 16)
                hrows_v[j, sl] = hrows_v[j, sl] * dn + attr_v[j, sl]
            return carry

        lax.fori_loop(0, 128, wrow, 0)
        pltpu.sync_copy(hrows_v, out.at[pl.ds(cid * NP + r0, 128)])


def kernel(x, edge_index, edge_attr, W_mean, b_mean, W_std, b_std):
    x_pad = jnp.pad(x, ((0, NP - N), (0, 0)))
    W_all = jnp.stack([W_mean, W_std])
    b_all = jnp.stack([b_mean, b_std])[:, None, :]
    h_all = _linear(x_pad, W_all, b_all)
    h_flat = h_all.reshape(2 * NP, D)
    rows2d = edge_index[0].reshape(NCHUNKS, CHUNK)
    cols2d = edge_index[1].reshape(NCHUNKS, CHUNK)
    radj2d = jnp.concatenate([rows2d, rows2d + NP], axis=0)
    zrows = jnp.zeros((CHUNK, D), jnp.float32)
    zdeg = jnp.zeros((DEGR, L), jnp.float32)
    idrows = jnp.arange(5 * CHUNK, dtype=jnp.int32).reshape(5, CHUNK)
    out_flat = _sc_kernel(
        rows2d, cols2d, radj2d, edge_attr, h_flat, zrows, zdeg, idrows
    )
    out = out_flat.reshape(2, NP, D)
    return out[0, :N], out[1, :N]
